# Initial kernel scaffold; baseline (speedup 1.0000x reference)
import jax
import jax.numpy as jnp
from jax.experimental import pallas as pl


def kernel(token_ids, W):
    raise NotImplementedError("write your pallas kernel here")


# speedup vs baseline: 1.0992x; 1.0992x over previous
import functools

import jax
import jax.numpy as jnp
from jax import lax
from jax.experimental import pallas as pl
from jax.experimental.pallas import tpu as pltpu
from jax.experimental.pallas import tpu_sc as plsc

DIM = 32
B_TOTAL = 16384 * 50
NUM_WORKERS = 32
B_PER_W = B_TOTAL // NUM_WORKERS
CHUNK = 1280
N_CHUNKS = B_PER_W // CHUNK


def _emb_body(w_hbm, idx_hbm, out_hbm, idx_v, rows_v, sem):
    wid = lax.axis_index("s") * 2 + lax.axis_index("c")
    base = wid * B_PER_W

    def body(i, carry):
        off = base + i * CHUNK
        pltpu.sync_copy(idx_hbm.at[pl.ds(off, CHUNK)], idx_v)
        pltpu.async_copy(w_hbm.at[idx_v], rows_v, sem).wait()
        pltpu.sync_copy(rows_v, out_hbm.at[pl.ds(off, CHUNK)])
        return carry

    lax.fori_loop(0, N_CHUNKS, body, 0)


@jax.jit
def _embed(W, idx_flat):
    mesh = plsc.VectorSubcoreMesh(core_axis_name="c", subcore_axis_name="s")
    f = functools.partial(
        pl.kernel,
        mesh=mesh,
        out_type=jax.ShapeDtypeStruct((B_TOTAL, DIM), jnp.float32),
        scratch_types=[
            pltpu.VMEM((CHUNK,), jnp.int32),
            pltpu.VMEM((CHUNK, DIM), jnp.float32),
            pltpu.SemaphoreType.DMA,
        ],
        compiler_params=pltpu.CompilerParams(use_tc_tiling_on_sc=False),
    )(_emb_body)
    return f(W, idx_flat)


def kernel(token_ids, W):
    idx_flat = token_ids.reshape(-1).astype(jnp.int32)
    out = _embed(W, idx_flat)
    return out.reshape(token_ids.shape + (DIM,))


# speedup vs baseline: 1.1120x; 1.0117x over previous
import functools

import jax
import jax.numpy as jnp
from jax import lax
from jax.experimental import pallas as pl
from jax.experimental.pallas import tpu as pltpu
from jax.experimental.pallas import tpu_sc as plsc

DIM = 32
B_TOTAL = 16384 * 50
NUM_WORKERS = 32
B_PER_W = B_TOTAL // NUM_WORKERS
CHUNK = 1280
N_CHUNKS = B_PER_W // CHUNK


def _emb_body(w_hbm, idx_hbm, out_hbm,
              idx0, idx1, rows0, rows1,
              s_i0, s_i1, s_g0, s_g1, s_o0, s_o1):
    idx_v = [idx0, idx1]
    rows_v = [rows0, rows1]
    s_i = [s_i0, s_i1]
    s_g = [s_g0, s_g1]
    s_o = [s_o0, s_o1]

    wid = lax.axis_index("s") * 2 + lax.axis_index("c")
    base = wid * B_PER_W

    def idx_copy(chunk, b):
        return pltpu.make_async_copy(
            idx_hbm.at[pl.ds(base + chunk * CHUNK, CHUNK)], idx_v[b], s_i[b])

    def gather(b):
        return pltpu.make_async_copy(w_hbm.at[idx_v[b]], rows_v[b], s_g[b])

    def writeback(chunk, b):
        return pltpu.make_async_copy(
            rows_v[b], out_hbm.at[pl.ds(base + chunk * CHUNK, CHUNK)], s_o[b])

    idx_copy(0, 0).start()
    idx_copy(1, 1).start()
    idx_copy(0, 0).wait()
    gather(0).start()

    def outer(g, carry):
        idx_copy(g + 1, 1).wait()

        @pl.when(g >= 1)
        def _():
            writeback(g - 1, 1).wait()

        gather(1).start()
        gather(0).wait()
        writeback(g, 0).start()

        @pl.when(g < N_CHUNKS - 2)
        def _():
            idx_copy(g + 2, 0).start()

        @pl.when(g < N_CHUNKS - 2)
        def _():
            idx_copy(g + 2, 0).wait()
            writeback(g, 0).wait()
            gather(0).start()

        gather(1).wait()
        writeback(g + 1, 1).start()

        @pl.when(g < N_CHUNKS - 2)
        def _():
            idx_copy(g + 3, 1).start()

        return carry

    lax.fori_loop(0, N_CHUNKS // 2, lambda j, c: outer(j * 2, c), 0,
                  unroll=False)

    writeback(N_CHUNKS - 2, 0).wait()
    writeback(N_CHUNKS - 1, 1).wait()


@jax.jit
def _embed(W, idx_flat):
    mesh = plsc.VectorSubcoreMesh(core_axis_name="c", subcore_axis_name="s")
    f = functools.partial(
        pl.kernel,
        mesh=mesh,
        out_type=jax.ShapeDtypeStruct((B_TOTAL, DIM), jnp.float32),
        scratch_types=[
            pltpu.VMEM((CHUNK,), jnp.int32),
            pltpu.VMEM((CHUNK,), jnp.int32),
            pltpu.VMEM((CHUNK, DIM), jnp.float32),
            pltpu.VMEM((CHUNK, DIM), jnp.float32),
            pltpu.SemaphoreType.DMA,
            pltpu.SemaphoreType.DMA,
            pltpu.SemaphoreType.DMA,
            pltpu.SemaphoreType.DMA,
            pltpu.SemaphoreType.DMA,
            pltpu.SemaphoreType.DMA,
        ],
        compiler_params=pltpu.CompilerParams(use_tc_tiling_on_sc=False),
    )(_emb_body)
    return f(W, idx_flat)


def kernel(token_ids, W):
    idx_flat = token_ids.reshape(-1).astype(jnp.int32)
    out = _embed(W, idx_flat)
    return out.reshape(token_ids.shape + (DIM,))


# speedup vs baseline: 1.1123x; 1.0002x over previous
import functools

import jax
import jax.numpy as jnp
from jax import lax
from jax.experimental import pallas as pl
from jax.experimental.pallas import tpu as pltpu
from jax.experimental.pallas import tpu_sc as plsc

DIM = 32
B_TOTAL = 16384 * 50
NUM_WORKERS = 32
B_PER_W = B_TOTAL // NUM_WORKERS
CHUNK = 1280
N_CHUNKS = B_PER_W // CHUNK
N_STREAMS = 4
SUB = CHUNK // N_STREAMS


def _emb_body(w_hbm, idx_hbm, out_hbm,
              idx0, idx1, rows0, rows1,
              s_i0, s_i1, s_g0, s_g1, s_o0, s_o1):
    idx_v = [idx0, idx1]
    rows_v = [rows0, rows1]
    s_i = [s_i0, s_i1]
    s_g = [s_g0, s_g1]
    s_o = [s_o0, s_o1]

    wid = lax.axis_index("s") * 2 + lax.axis_index("c")
    base = wid * B_PER_W

    def idx_copy(chunk, b):
        return pltpu.make_async_copy(
            idx_hbm.at[pl.ds(base + chunk * CHUNK, CHUNK)], idx_v[b], s_i[b])

    def _gather_descs(b):
        return [pltpu.make_async_copy(
                    w_hbm.at[idx_v[b].at[pl.ds(j * SUB, SUB)]],
                    rows_v[b].at[pl.ds(j * SUB, SUB)], s_g[b])
                for j in range(N_STREAMS)]

    class gather:
        def __init__(self, b):
            self.b = b

        def start(self):
            for d in _gather_descs(self.b):
                d.start()

        def wait(self):
            for d in _gather_descs(self.b):
                d.wait()

    def writeback(chunk, b):
        return pltpu.make_async_copy(
            rows_v[b], out_hbm.at[pl.ds(base + chunk * CHUNK, CHUNK)], s_o[b])

    idx_copy(0, 0).start()
    idx_copy(1, 1).start()
    idx_copy(0, 0).wait()
    gather(0).start()

    def outer(g, carry):
        idx_copy(g + 1, 1).wait()

        @pl.when(g >= 1)
        def _():
            writeback(g - 1, 1).wait()

        gather(1).start()
        gather(0).wait()
        writeback(g, 0).start()

        @pl.when(g < N_CHUNKS - 2)
        def _():
            idx_copy(g + 2, 0).start()

        @pl.when(g < N_CHUNKS - 2)
        def _():
            idx_copy(g + 2, 0).wait()
            writeback(g, 0).wait()
            gather(0).start()

        gather(1).wait()
        writeback(g + 1, 1).start()

        @pl.when(g < N_CHUNKS - 2)
        def _():
            idx_copy(g + 3, 1).start()

        return carry

    lax.fori_loop(0, N_CHUNKS // 2, lambda j, c: outer(j * 2, c), 0,
                  unroll=False)

    writeback(N_CHUNKS - 2, 0).wait()
    writeback(N_CHUNKS - 1, 1).wait()


@jax.jit
def _embed(W, idx_flat):
    mesh = plsc.VectorSubcoreMesh(core_axis_name="c", subcore_axis_name="s")
    f = functools.partial(
        pl.kernel,
        mesh=mesh,
        out_type=jax.ShapeDtypeStruct((B_TOTAL, DIM), jnp.float32),
        scratch_types=[
            pltpu.VMEM((CHUNK,), jnp.int32),
            pltpu.VMEM((CHUNK,), jnp.int32),
            pltpu.VMEM((CHUNK, DIM), jnp.float32),
            pltpu.VMEM((CHUNK, DIM), jnp.float32),
            pltpu.SemaphoreType.DMA,
            pltpu.SemaphoreType.DMA,
            pltpu.SemaphoreType.DMA,
            pltpu.SemaphoreType.DMA,
            pltpu.SemaphoreType.DMA,
            pltpu.SemaphoreType.DMA,
        ],
        compiler_params=pltpu.CompilerParams(use_tc_tiling_on_sc=False),
    )(_emb_body)
    return f(W, idx_flat)


def kernel(token_ids, W):
    idx_flat = token_ids.reshape(-1).astype(jnp.int32)
    out = _embed(W, idx_flat)
    return out.reshape(token_ids.shape + (DIM,))
